# Initial kernel scaffold; baseline (speedup 1.0000x reference)
import jax
import jax.numpy as jnp
from jax.experimental import pallas as pl


def kernel(objectness, pred_bbox_deltas, anchors):
    raise NotImplementedError("write your pallas kernel here")


# speedup vs baseline: 1.5920x; 1.5920x over previous
import functools

import jax
import jax.numpy as jnp
import numpy as np
from jax.experimental import pallas as pl
from jax.experimental.pallas import tpu as pltpu

PRE = 1000
APAD = 20480
ROWS = 160
CAP = 1024
NMS_T = 0.7
MINSZ = 0.001
IMGH = 512.0
IMGW = 512.0
BCLIP = float(np.log(1000.0 / 16.0))


def _row1024(x):
    return jnp.concatenate([x[s:s + 1, :] for s in range(8)], axis=1)


def _col1024(x):
    xt = jnp.transpose(x)
    return jnp.concatenate([xt[:, s:s + 1] for s in range(8)], axis=0)


def _stack8(r):
    return jnp.concatenate([r[:, 128 * s:128 * (s + 1)] for s in range(8)],
                           axis=0)


def _rpn_body(obj_ref, del_ref, anc_ref, boxes_ref, scores_ref):
    obj = obj_ref[0]
    ax1 = anc_ref[0]
    ay1 = anc_ref[1]
    ax2 = anc_ref[2]
    ay2 = anc_ref[3]
    w = ax2 - ax1
    h = ay2 - ay1
    cx = ax1 + 0.5 * w
    cy = ay1 + 0.5 * h
    dx = del_ref[0, 0]
    dy = del_ref[0, 1]
    dw = jnp.minimum(del_ref[0, 2], BCLIP)
    dh = jnp.minimum(del_ref[0, 3], BCLIP)
    pcx = dx * w + cx
    pcy = dy * h + cy
    pw = jnp.exp(dw) * w
    ph = jnp.exp(dh) * h
    x1 = jnp.clip(pcx - 0.5 * pw, 0.0, IMGW)
    y1 = jnp.clip(pcy - 0.5 * ph, 0.0, IMGH)
    x2 = jnp.clip(pcx + 0.5 * pw, 0.0, IMGW)
    y2 = jnp.clip(pcy + 0.5 * ph, 0.0, IMGH)

    bits = jax.lax.bitcast_convert_type(obj, jnp.int32)
    skey = jnp.where(bits < 0, bits ^ jnp.int32(0x7FFFFFFF), bits)
    kf = jnp.float32(PRE)
    n_ge0 = jnp.sum((skey >= 0).astype(jnp.float32))
    p0 = n_ge0 >= kf
    lo0 = jnp.where(p0, jnp.int32(0), jnp.int32(-2147483648))
    hi0 = jnp.where(p0, jnp.int32(2147483647), jnp.int32(-1))

    def bs_body(_, c):
        lo, hi = c
        d = hi - lo
        mid = lo + (d >> 1) + (d & 1)
        cnt = jnp.sum((skey >= mid).astype(jnp.float32))
        p = cnt >= kf
        return (jnp.where(p, mid, lo), jnp.where(p, hi, mid - 1))

    lo, _ = jax.lax.fori_loop(0, 31, bs_body, (lo0, hi0))
    mask_ge = skey >= lo
    n_sel = jnp.sum(mask_ge.astype(jnp.float32))

    mf = mask_ge.astype(jnp.float32)
    u128 = (jax.lax.broadcasted_iota(jnp.int32, (128, 128), 0) <=
            jax.lax.broadcasted_iota(jnp.int32, (128, 128), 1)
            ).astype(jnp.float32)
    cs = jnp.dot(mf, u128, preferred_element_type=jnp.float32)
    row_tot = cs[:, 127:128]
    lt160 = (jax.lax.broadcasted_iota(jnp.int32, (ROWS, ROWS), 1) <
             jax.lax.broadcasted_iota(jnp.int32, (ROWS, ROWS), 0)
             ).astype(jnp.float32)
    offs = jnp.dot(lt160, row_tot, preferred_element_type=jnp.float32)
    pos = cs + offs - 1.0
    pos = jnp.where(mask_ge, pos, -1.0)

    iota_r = jax.lax.broadcasted_iota(jnp.int32, (1, CAP), 1).astype(jnp.float32)
    objf = jnp.where(mask_ge, obj, 0.0)
    acc = jnp.zeros((5, CAP), jnp.float32)
    for g in range(20):
        sl = slice(8 * g, 8 * g + 8)
        pcol = _col1024(pos[sl])
        oh = (pcol == iota_r).astype(jnp.float32)
        dflat = jnp.concatenate([
            _row1024(objf[sl]),
            _row1024(x1[sl]), _row1024(y1[sl]),
            _row1024(x2[sl]), _row1024(y2[sl])], axis=0)
        acc = acc + jnp.dot(dflat, oh,
                            preferred_element_type=jnp.float32,
                            precision=jax.lax.Precision.HIGHEST)

    srow = acc[0:1, :]
    validc = iota_r < n_sel
    seff = jnp.where(validc, srow, -jnp.inf)
    scol = _col1024(_stack8(seff))
    icol = jax.lax.broadcasted_iota(jnp.int32, (CAP, 1), 0).astype(jnp.float32)
    m = ((seff > scol).astype(jnp.float32) +
         ((seff == scol) & (iota_r < icol)).astype(jnp.float32))
    rank = jnp.sum(m, axis=1, keepdims=True)
    oh2 = (rank == iota_r).astype(jnp.float32)
    srt = jnp.dot(acc, oh2, preferred_element_type=jnp.float32,
                  precision=jax.lax.Precision.HIGHEST)

    sc = _stack8(srt[0:1, :])
    bx1 = _stack8(srt[1:2, :])
    by1 = _stack8(srt[2:3, :])
    bx2 = _stack8(srt[3:4, :])
    by2 = _stack8(srt[4:5, :])
    isl = (jax.lax.broadcasted_iota(jnp.int32, (8, 128), 0) * 128 +
           jax.lax.broadcasted_iota(jnp.int32, (8, 128), 1)).astype(jnp.float32)
    small = ((bx2 - bx1) < MINSZ) | ((by2 - by1) < MINSZ)
    keep0 = ((~small) & (isl < float(PRE))).astype(jnp.float32)
    area = (bx2 - bx1) * (by2 - by1)

    def nms_body(i, keep):
        fi = i.astype(jnp.float32)
        cur = (isl == fi).astype(jnp.float32)
        ki = jnp.sum(keep * cur)
        cx1 = jnp.sum(bx1 * cur)
        cy1 = jnp.sum(by1 * cur)
        cx2 = jnp.sum(bx2 * cur)
        cy2 = jnp.sum(by2 * cur)
        ca = jnp.sum(area * cur)
        iw = jnp.maximum(jnp.minimum(cx2, bx2) - jnp.maximum(cx1, bx1), 0.0)
        ih = jnp.maximum(jnp.minimum(cy2, by2) - jnp.maximum(cy1, by1), 0.0)
        inter = iw * ih
        iou = inter / (ca + area - inter + 1e-9)
        sup = ((iou > NMS_T) & (isl > fi)).astype(jnp.float32)
        return keep * (1.0 - sup * ki)

    keep = jax.lax.fori_loop(0, PRE, nms_body, keep0)

    fsc = jnp.where(keep > 0.5, sc, -jnp.inf)
    scores_ref[0] = _row1024(fsc)
    boxes_ref[0] = jnp.concatenate([
        _row1024(bx1 * keep), _row1024(by1 * keep),
        _row1024(bx2 * keep), _row1024(by2 * keep)], axis=0)


@jax.jit
def kernel(objectness, pred_bbox_deltas, anchors):
    b, a = objectness.shape
    objp = jnp.full((b, APAD), -jnp.inf, jnp.float32).at[:, :a].set(objectness)
    obj3 = objp.reshape(b, ROWS, 128)
    dpad = jnp.zeros((b, APAD, 4), jnp.float32).at[:, :a, :].set(
        pred_bbox_deltas)
    d4 = dpad.transpose(0, 2, 1).reshape(b, 4, ROWS, 128)
    apad = jnp.zeros((APAD, 4), jnp.float32).at[:a, :].set(anchors)
    a4 = jnp.transpose(apad).reshape(4, ROWS, 128)
    boxes_p, scores_p = pl.pallas_call(
        _rpn_body,
        grid=(b,),
        in_specs=[
            pl.BlockSpec((1, ROWS, 128), lambda i: (i, 0, 0)),
            pl.BlockSpec((1, 4, ROWS, 128), lambda i: (i, 0, 0, 0)),
            pl.BlockSpec((4, ROWS, 128), lambda i: (0, 0, 0)),
        ],
        out_specs=[
            pl.BlockSpec((1, 4, CAP), lambda i: (i, 0, 0)),
            pl.BlockSpec((1, 1, CAP), lambda i: (i, 0, 0)),
        ],
        out_shape=[
            jax.ShapeDtypeStruct((b, 4, CAP), jnp.float32),
            jax.ShapeDtypeStruct((b, 1, CAP), jnp.float32),
        ],
    )(obj3, d4, a4)
    final_boxes = boxes_p[:, :, :PRE].transpose(0, 2, 1)
    final_scores = scores_p[:, 0, :PRE]
    return final_boxes, final_scores
